# baseline (speedup 1.0000x reference)
import functools

import jax
import jax.numpy as jnp
from jax import lax
from jax.experimental import pallas as pl
from jax.experimental.pallas import tpu as pltpu
from jax.experimental.pallas import tpu_sc as plsc

N_IN_ROWS = 100000
N_OUT_ROWS = 200000
NK = 27
NP = 20000
C = 128
KP = NK * NP

NCORE = 2
NSUB = 16
NW = NCORE * NSUB

ROWS_PER_W = 16896
KP_PAD = NW * ROWS_PER_W
GCHUNK = 384
NGCH = ROWS_PER_W // GCHUNK
GSTREAMS = GCHUNK // 128
IDX_ROWS_W = ROWS_PER_W // 128

MM_BLK = 2000

SCHUNK = 512
MS_PER_TILE = KP_PAD // NSUB
CS = 16
NSLAB = C // CS
HALF = N_OUT_ROWS // 2
TRASH = 1024
SLAB_R = HALF + TRASH
ZROWS = SLAB_R // NSUB
EXP_R = HALF // NSUB
LCAP = MS_PER_TILE + 1024
NST = MS_PER_TILE // 1024

_MESH = plsc.VectorSubcoreMesh(core_axis_name="c", subcore_axis_name="s")
_SC_PARAMS = pltpu.CompilerParams(use_tc_tiling_on_sc=False)
_SC_PARAMS_NL = pltpu.CompilerParams(use_tc_tiling_on_sc=False,
                                     needs_layout_passes=False)


def _gather_body(x_hbm, im_hbm, g_hbm, idx_v, rows_a, rows_b,
                 sem_ga, sem_gb, sem_wa, sem_wb):
    wid = lax.axis_index("s") * NCORE + lax.axis_index("c")
    pltpu.sync_copy(im_hbm.at[pl.ds(wid * IDX_ROWS_W, IDX_ROWS_W)], idx_v)
    base = wid * ROWS_PER_W

    def issue_g(ci, rows_v, sem):
        for j in range(GSTREAMS):
            pltpu.async_copy(
                x_hbm.at[idx_v.at[ci * GSTREAMS + j]],
                rows_v.at[pl.ds(j * 128, 128)],
                sem,
            )

    def drain_g(rows_v, sem):
        for j in range(GSTREAMS):
            pltpu.make_async_copy(
                x_hbm.at[pl.ds(0, 128)],
                rows_v.at[pl.ds(j * 128, 128)],
                sem,
            ).wait()

    def issue_w(ci, rows_v, sem):
        pltpu.async_copy(rows_v, g_hbm.at[pl.ds(base + ci * GCHUNK, GCHUNK)], sem)

    def drain_w(rows_v, sem):
        pltpu.make_async_copy(rows_v, g_hbm.at[pl.ds(0, GCHUNK)], sem).wait()

    issue_g(0, rows_a, sem_ga)

    def pair(i, carry):
        c = 2 * i
        drain_g(rows_a, sem_ga)

        @pl.when(i > 0)
        def _():
            drain_w(rows_b, sem_wb)

        issue_g(c + 1, rows_b, sem_gb)
        issue_w(c, rows_a, sem_wa)
        drain_g(rows_b, sem_gb)
        drain_w(rows_a, sem_wa)

        @pl.when(i < NGCH // 2 - 1)
        def _():
            issue_g(c + 2, rows_a, sem_ga)

        issue_w(c + 1, rows_b, sem_wb)
        return carry

    lax.fori_loop(0, NGCH // 2, pair, 0)
    drain_w(rows_b, sem_wb)


@functools.partial(
    pl.kernel,
    out_type=jax.ShapeDtypeStruct((KP_PAD, C), jnp.float32),
    mesh=_MESH,
    scratch_types=[
        pltpu.VMEM((IDX_ROWS_W, 128), jnp.int32),
        pltpu.VMEM((GCHUNK, C), jnp.float32),
        pltpu.VMEM((GCHUNK, C), jnp.float32),
        pltpu.SemaphoreType.DMA,
        pltpu.SemaphoreType.DMA,
        pltpu.SemaphoreType.DMA,
        pltpu.SemaphoreType.DMA,
    ],
    compiler_params=_SC_PARAMS,
)
def _gather_call(x_hbm, im_hbm, g_hbm, idx_v, rows_a, rows_b,
                 sem_ga, sem_gb, sem_wa, sem_wb):
    _gather_body(x_hbm, im_hbm, g_hbm, idx_v, rows_a, rows_b,
                 sem_ga, sem_gb, sem_wa, sem_wb)


def _mm_body(g_ref, w_ref, o_ref):
    o_ref[...] = jnp.dot(g_ref[...], w_ref[0], preferred_element_type=jnp.float32)


def _matmul(g, w):
    return pl.pallas_call(
        _mm_body,
        grid=(NK, NP // MM_BLK),
        in_specs=[
            pl.BlockSpec((MM_BLK, C), lambda k, b: (k * (NP // MM_BLK) + b, 0)),
            pl.BlockSpec((1, C, C), lambda k, b: (k, 0, 0)),
        ],
        out_specs=pl.BlockSpec((MM_BLK, C), lambda k, b: (k * (NP // MM_BLK) + b, 0)),
        out_shape=jax.ShapeDtypeStruct((KP_PAD, C), jnp.float32),
    )(g, w)


def _partition_body(om_hbm, mlist_hbm, olist_hbm, cnt_hbm,
                    stage_v, mbuf, obuf, cvec):
    cid = lax.axis_index("c")
    sid = lax.axis_index("s")
    rbase = cid * HALF
    lane = lax.iota(jnp.int32, 16)

    def staging(st, off):
        row0 = sid * (MS_PER_TILE // 128) + st * 8
        pltpu.sync_copy(om_hbm.at[pl.ds(row0, 8)], stage_v)
        gbase = sid * MS_PER_TILE + st * 1024
        for j in range(8):
            for i in range(8):
                v = stage_v[j, pl.ds(i * 16, 16)]
                r = v - rbase
                ok = (r >= 0) & (r < HALF)
                gidx = gbase + j * 128 + i * 16 + lane
                pos = plsc.cumsum(ok.astype(jnp.int32))
                dest = jnp.where(ok, off + pos - 1, LCAP + lane)
                plsc.store_scatter(mbuf, [dest], gidx)
                plsc.store_scatter(obuf, [dest], r)
                off = off + jnp.max(pos)
        return off

    off = lax.fori_loop(0, NST, staging, jnp.int32(0))
    for t in range(64):
        m_pad = lane + (t % 8) * 16
        o_pad = HALF + lane + (t % 8) * 16
        mbuf[pl.ds(off + t * 16, 16)] = m_pad
        obuf[pl.ds(off + t * 16, 16)] = o_pad
    padded = ((off + 1023) >> 10) << 10
    cvec[:] = jnp.full((16,), 0, jnp.int32) + padded
    pltpu.sync_copy(cvec, cnt_hbm.at[cid, sid])
    lbase = (cid * NSUB + sid) * LCAP
    pltpu.sync_copy(mbuf.at[pl.ds(0, LCAP)], mlist_hbm.at[pl.ds(lbase, LCAP)])
    pltpu.sync_copy(obuf.at[pl.ds(0, LCAP)], olist_hbm.at[pl.ds(lbase, LCAP)])


@functools.partial(
    pl.kernel,
    out_type=(
        jax.ShapeDtypeStruct((NCORE * NSUB * LCAP,), jnp.int32),
        jax.ShapeDtypeStruct((NCORE * NSUB * LCAP,), jnp.int32),
        jax.ShapeDtypeStruct((NCORE, NSUB, 16), jnp.int32),
    ),
    mesh=_MESH,
    scratch_types=[
        pltpu.VMEM((8, 128), jnp.int32),
        pltpu.VMEM((LCAP + 16,), jnp.int32),
        pltpu.VMEM((LCAP + 16,), jnp.int32),
        pltpu.VMEM((16,), jnp.int32),
    ],
    compiler_params=_SC_PARAMS_NL,
)
def _partition_call(om_hbm, mlist_hbm, olist_hbm, cnt_hbm,
                    stage_v, mbuf, obuf, cvec):
    _partition_body(om_hbm, mlist_hbm, olist_hbm, cnt_hbm,
                    stage_v, mbuf, obuf, cvec)


def _scatter_body(m2_hbm, mlist_hbm, olist_hbm, cnt_hbm, out_hbm, slab,
                  mst_a, mst_b, ost_a, ost_b, ost2_a, ost2_b,
                  data_a, data_b, cvec, zero_v,
                  sem_l_a, sem_l_b, sem_g_a, sem_g_b, sem_add_a, sem_add_b):
    cid = lax.axis_index("c")
    sid = lax.axis_index("s")
    rbase = cid * HALF
    lbase = (cid * NSUB + sid) * LCAP

    def zfill(i, carry):
        zero_v[i, :] = jnp.zeros((CS,), jnp.float32)
        return carry

    lax.fori_loop(0, SCHUNK, zfill, 0)
    zfull = ZROWS // SCHUNK
    zrem = ZROWS - zfull * SCHUNK

    pltpu.sync_copy(cnt_hbm.at[cid, sid], cvec)
    n512 = jnp.max(cvec[...]) >> 9

    def issue_lists(g, mst_v, ost_v, sem):
        pltpu.async_copy(mlist_hbm.at[pl.ds(lbase + g * 512, 512)], mst_v, sem)
        pltpu.async_copy(olist_hbm.at[pl.ds(lbase + g * 512, 512)], ost_v, sem)

    def drain_lists(mst_v, ost_v, sem):
        pltpu.make_async_copy(mlist_hbm.at[pl.ds(0, 512)], mst_v, sem).wait()
        pltpu.make_async_copy(olist_hbm.at[pl.ds(0, 512)], ost_v, sem).wait()

    def process(s, mst_v, ost_v, ost2_v, data_v, sem_g, sem_add):
        for k in range(32):
            m = mst_v[pl.ds(k * 16, 16)]
            mst_v[pl.ds(k * 16, 16)] = (m << 3) + s
            ost2_v[k // 8, pl.ds((k % 8) * 16, 16)] = ost_v[pl.ds(k * 16, 16)]
        descs = []
        for j in range(4):
            descs.append(pltpu.async_copy(
                m2_hbm.at[mst_v.at[pl.ds(j * 128, 128)]],
                data_v.at[pl.ds(j * 128, 128)],
                sem_g,
            ))
        for d in descs:
            d.wait()
        descs = []
        for j in range(4):
            descs.append(pltpu.async_copy(
                data_v.at[pl.ds(j * 128, 128)],
                slab.at[ost2_v.at[j]],
                sem_add,
                add=True,
            ))
        for d in descs:
            d.wait()

    for s in range(NSLAB):
        for zi in range(zfull):
            pltpu.sync_copy(zero_v, slab.at[pl.ds(sid * ZROWS + zi * SCHUNK, SCHUNK)])
        pltpu.sync_copy(
            zero_v.at[pl.ds(0, zrem)],
            slab.at[pl.ds(sid * ZROWS + zfull * SCHUNK, zrem)],
        )
        plsc.subcore_barrier()

        issue_lists(0, mst_a, ost_a, sem_l_a)

        def pair(i, carry):
            issue_lists(2 * i + 1, mst_b, ost_b, sem_l_b)
            drain_lists(mst_a, ost_a, sem_l_a)
            process(s, mst_a, ost_a, ost2_a, data_a, sem_g_a, sem_add_a)

            @pl.when(2 * i + 2 < n512)
            def _():
                issue_lists(2 * i + 2, mst_a, ost_a, sem_l_a)

            drain_lists(mst_b, ost_b, sem_l_b)
            process(s, mst_b, ost_b, ost2_b, data_b, sem_g_b, sem_add_b)
            return carry

        lax.fori_loop(0, n512 >> 1, pair, 0)
        plsc.subcore_barrier()
        pltpu.sync_copy(
            slab.at[pl.ds(sid * EXP_R, EXP_R)],
            out_hbm.at[pl.ds(rbase + sid * EXP_R, EXP_R), pl.ds(s * CS, CS)],
        )
        plsc.subcore_barrier()


@functools.partial(
    pl.kernel,
    out_type=jax.ShapeDtypeStruct((N_OUT_ROWS, C), jnp.float32),
    mesh=_MESH,
    scratch_types=[
        pltpu.VMEM_SHARED((SLAB_R, CS), jnp.float32),
        pltpu.VMEM((SCHUNK,), jnp.int32),
        pltpu.VMEM((SCHUNK,), jnp.int32),
        pltpu.VMEM((SCHUNK,), jnp.int32),
        pltpu.VMEM((SCHUNK,), jnp.int32),
        pltpu.VMEM((SCHUNK // 128, 128), jnp.int32),
        pltpu.VMEM((SCHUNK // 128, 128), jnp.int32),
        pltpu.VMEM((SCHUNK, CS), jnp.float32),
        pltpu.VMEM((SCHUNK, CS), jnp.float32),
        pltpu.VMEM((16,), jnp.int32),
        pltpu.VMEM((SCHUNK, CS), jnp.float32),
        pltpu.SemaphoreType.DMA,
        pltpu.SemaphoreType.DMA,
        pltpu.SemaphoreType.DMA,
        pltpu.SemaphoreType.DMA,
        pltpu.SemaphoreType.DMA,
        pltpu.SemaphoreType.DMA,
    ],
    compiler_params=_SC_PARAMS_NL,
)
def _scatter_call(m2_hbm, mlist_hbm, olist_hbm, cnt_hbm, out_hbm, slab,
                  mst_a, mst_b, ost_a, ost_b, ost2_a, ost2_b,
                  data_a, data_b, cvec, zero_v,
                  sem_l_a, sem_l_b, sem_g_a, sem_g_b, sem_add_a, sem_add_b):
    _scatter_body(m2_hbm, mlist_hbm, olist_hbm, cnt_hbm, out_hbm, slab,
                  mst_a, mst_b, ost_a, ost_b, ost2_a, ost2_b,
                  data_a, data_b, cvec, zero_v,
                  sem_l_a, sem_l_b, sem_g_a, sem_g_b, sem_add_a, sem_add_b)


def kernel(x, in_map, out_map, kernel):
    w = kernel
    pad = KP_PAD - KP
    im = in_map.reshape(-1).astype(jnp.int32)
    om = out_map.reshape(-1).astype(jnp.int32)
    pad_in = (jnp.arange(pad, dtype=jnp.int32) * 149) % N_IN_ROWS
    pad_out = N_OUT_ROWS + jnp.arange(pad, dtype=jnp.int32)
    im_p = jnp.concatenate([im, pad_in]).reshape(KP_PAD // 128, 128)
    om_p = jnp.concatenate([om, pad_out]).reshape(KP_PAD // 128, 128)

    mlist, olist, cnts = _partition_call(om_p)
    g = _gather_call(x, im_p)
    msgs = _matmul(g, w)
    m2 = msgs.reshape(KP_PAD * 8, CS)
    return _scatter_call(m2, mlist, olist, cnts)


# speedup vs baseline: 1.0033x; 1.0033x over previous
import functools

import jax
import jax.numpy as jnp
from jax import lax
from jax.experimental import pallas as pl
from jax.experimental.pallas import tpu as pltpu
from jax.experimental.pallas import tpu_sc as plsc

N_IN_ROWS = 100000
N_OUT_ROWS = 200000
NK = 27
NP = 20000
C = 128
KP = NK * NP

NCORE = 2
NSUB = 16
NW = NCORE * NSUB

ROWS_PER_W = 16896
KP_PAD = NW * ROWS_PER_W
GCHUNK = 384
NGCH = ROWS_PER_W // GCHUNK
GSTREAMS = GCHUNK // 128
IDX_ROWS_W = ROWS_PER_W // 128

MM_BLK = 2000

SCHUNK = 512
MS_PER_TILE = KP_PAD // NSUB
CS = 16
NSLAB = C // CS
HALF = N_OUT_ROWS // 2
TRASH = 1024
SLAB_R = HALF + TRASH
ZROWS = SLAB_R // NSUB
EXP_R = HALF // NSUB
LCAP = MS_PER_TILE + 1024
NST = MS_PER_TILE // 1024

_MESH = plsc.VectorSubcoreMesh(core_axis_name="c", subcore_axis_name="s")
_SC_PARAMS = pltpu.CompilerParams(use_tc_tiling_on_sc=False)
_SC_PARAMS_NL = pltpu.CompilerParams(use_tc_tiling_on_sc=False,
                                     needs_layout_passes=False)


def _gather_body(x_hbm, im_hbm, g_hbm, idx_v, rows_a, rows_b,
                 sem_ga, sem_gb, sem_wa, sem_wb):
    wid = lax.axis_index("s") * NCORE + lax.axis_index("c")
    pltpu.sync_copy(im_hbm.at[pl.ds(wid * IDX_ROWS_W, IDX_ROWS_W)], idx_v)
    base = wid * ROWS_PER_W

    def issue_g(ci, rows_v, sem):
        for j in range(GSTREAMS):
            pltpu.async_copy(
                x_hbm.at[idx_v.at[ci * GSTREAMS + j]],
                rows_v.at[pl.ds(j * 128, 128)],
                sem,
            )

    def drain_g(rows_v, sem):
        for j in range(GSTREAMS):
            pltpu.make_async_copy(
                x_hbm.at[pl.ds(0, 128)],
                rows_v.at[pl.ds(j * 128, 128)],
                sem,
            ).wait()

    def issue_w(ci, rows_v, sem):
        pltpu.async_copy(rows_v, g_hbm.at[pl.ds(base + ci * GCHUNK, GCHUNK)], sem)

    def drain_w(rows_v, sem):
        pltpu.make_async_copy(rows_v, g_hbm.at[pl.ds(0, GCHUNK)], sem).wait()

    issue_g(0, rows_a, sem_ga)

    def pair(i, carry):
        c = 2 * i
        drain_g(rows_a, sem_ga)

        @pl.when(i > 0)
        def _():
            drain_w(rows_b, sem_wb)

        issue_g(c + 1, rows_b, sem_gb)
        issue_w(c, rows_a, sem_wa)
        drain_g(rows_b, sem_gb)
        drain_w(rows_a, sem_wa)

        @pl.when(i < NGCH // 2 - 1)
        def _():
            issue_g(c + 2, rows_a, sem_ga)

        issue_w(c + 1, rows_b, sem_wb)
        return carry

    lax.fori_loop(0, NGCH // 2, pair, 0)
    drain_w(rows_b, sem_wb)


@functools.partial(
    pl.kernel,
    out_type=jax.ShapeDtypeStruct((KP_PAD, C), jnp.float32),
    mesh=_MESH,
    scratch_types=[
        pltpu.VMEM((IDX_ROWS_W, 128), jnp.int32),
        pltpu.VMEM((GCHUNK, C), jnp.float32),
        pltpu.VMEM((GCHUNK, C), jnp.float32),
        pltpu.SemaphoreType.DMA,
        pltpu.SemaphoreType.DMA,
        pltpu.SemaphoreType.DMA,
        pltpu.SemaphoreType.DMA,
    ],
    compiler_params=_SC_PARAMS,
)
def _gather_call(x_hbm, im_hbm, g_hbm, idx_v, rows_a, rows_b,
                 sem_ga, sem_gb, sem_wa, sem_wb):
    _gather_body(x_hbm, im_hbm, g_hbm, idx_v, rows_a, rows_b,
                 sem_ga, sem_gb, sem_wa, sem_wb)


def _mm_body(g_ref, w_ref, o_ref):
    o_ref[...] = jnp.dot(g_ref[...], w_ref[0], preferred_element_type=jnp.float32)


def _matmul(g, w):
    return pl.pallas_call(
        _mm_body,
        grid=(NK, NP // MM_BLK),
        in_specs=[
            pl.BlockSpec((MM_BLK, C), lambda k, b: (k * (NP // MM_BLK) + b, 0)),
            pl.BlockSpec((1, C, C), lambda k, b: (k, 0, 0)),
        ],
        out_specs=pl.BlockSpec((MM_BLK, C), lambda k, b: (k * (NP // MM_BLK) + b, 0)),
        out_shape=jax.ShapeDtypeStruct((KP_PAD, C), jnp.float32),
    )(g, w)


def _partition_body(om_hbm, mlist_hbm, olist_hbm, cnt_hbm,
                    stage_v, mbuf, obuf, cvec):
    cid = lax.axis_index("c")
    sid = lax.axis_index("s")
    rbase = cid * HALF
    lane = lax.iota(jnp.int32, 16)

    def staging(st, off):
        row0 = sid * (MS_PER_TILE // 128) + st * 8
        pltpu.sync_copy(om_hbm.at[pl.ds(row0, 8)], stage_v)
        gbase = sid * MS_PER_TILE + st * 1024
        for j in range(8):
            for i in range(8):
                v = stage_v[j, pl.ds(i * 16, 16)]
                r = v - rbase
                ok = (r >= 0) & (r < HALF)
                gidx = gbase + j * 128 + i * 16 + lane
                pos = plsc.cumsum(ok.astype(jnp.int32))
                dest = jnp.where(ok, off + pos - 1, LCAP + lane)
                plsc.store_scatter(mbuf, [dest], gidx)
                plsc.store_scatter(obuf, [dest], r)
                off = off + jnp.max(pos)
        return off

    off = lax.fori_loop(0, NST, staging, jnp.int32(0))
    for t in range(64):
        m_pad = lane + (t % 8) * 16
        o_pad = HALF + lane + (t % 8) * 16
        mbuf[pl.ds(off + t * 16, 16)] = m_pad
        obuf[pl.ds(off + t * 16, 16)] = o_pad
    padded = ((off + 1023) >> 10) << 10
    cvec[:] = jnp.full((16,), 0, jnp.int32) + padded
    pltpu.sync_copy(cvec, cnt_hbm.at[cid, sid])
    lbase = (cid * NSUB + sid) * LCAP
    pltpu.sync_copy(mbuf.at[pl.ds(0, LCAP)], mlist_hbm.at[pl.ds(lbase, LCAP)])
    pltpu.sync_copy(obuf.at[pl.ds(0, LCAP)], olist_hbm.at[pl.ds(lbase, LCAP)])


@functools.partial(
    pl.kernel,
    out_type=(
        jax.ShapeDtypeStruct((NCORE * NSUB * LCAP,), jnp.int32),
        jax.ShapeDtypeStruct((NCORE * NSUB * LCAP,), jnp.int32),
        jax.ShapeDtypeStruct((NCORE, NSUB, 16), jnp.int32),
    ),
    mesh=_MESH,
    scratch_types=[
        pltpu.VMEM((8, 128), jnp.int32),
        pltpu.VMEM((LCAP + 16,), jnp.int32),
        pltpu.VMEM((LCAP + 16,), jnp.int32),
        pltpu.VMEM((16,), jnp.int32),
    ],
    compiler_params=_SC_PARAMS_NL,
)
def _partition_call(om_hbm, mlist_hbm, olist_hbm, cnt_hbm,
                    stage_v, mbuf, obuf, cvec):
    _partition_body(om_hbm, mlist_hbm, olist_hbm, cnt_hbm,
                    stage_v, mbuf, obuf, cvec)


def _scatter_body(m2_hbm, mlist_hbm, olist_hbm, cnt_hbm, out_hbm, slab,
                  mst_a, mst_b, ost_a, ost_b, ost2_a, ost2_b,
                  data_a, data_b, cvec, zero_v,
                  sem_l_a, sem_l_b, sem_g_a, sem_g_b, sem_add_a, sem_add_b):
    cid = lax.axis_index("c")
    sid = lax.axis_index("s")
    rbase = cid * HALF
    lbase = (cid * NSUB + sid) * LCAP

    def zfill(i, carry):
        zero_v[i, :] = jnp.zeros((CS,), jnp.float32)
        return carry

    lax.fori_loop(0, SCHUNK, zfill, 0)
    zfull = ZROWS // SCHUNK
    zrem = ZROWS - zfull * SCHUNK

    pltpu.sync_copy(cnt_hbm.at[cid, sid], cvec)
    n512 = jnp.max(cvec[...]) >> 9

    def issue_lists(g, mst_v, ost_v, sem):
        pltpu.async_copy(mlist_hbm.at[pl.ds(lbase + g * 512, 512)], mst_v, sem)
        pltpu.async_copy(olist_hbm.at[pl.ds(lbase + g * 512, 512)], ost_v, sem)

    def drain_lists(mst_v, ost_v, sem):
        pltpu.make_async_copy(mlist_hbm.at[pl.ds(0, 512)], mst_v, sem).wait()
        pltpu.make_async_copy(olist_hbm.at[pl.ds(0, 512)], ost_v, sem).wait()

    def process(s, mst_v, ost_v, ost2_v, data_v, sem_g, sem_add):
        for k in range(32):
            m = mst_v[pl.ds(k * 16, 16)]
            mst_v[pl.ds(k * 16, 16)] = (m << 3) + s
            ost2_v[k // 8, pl.ds((k % 8) * 16, 16)] = ost_v[pl.ds(k * 16, 16)]
        descs = []
        for j in range(4):
            descs.append(pltpu.async_copy(
                m2_hbm.at[mst_v.at[pl.ds(j * 128, 128)]],
                data_v.at[pl.ds(j * 128, 128)],
                sem_g,
            ))
        for d in descs:
            d.wait()
        descs = []
        for j in range(4):
            descs.append(pltpu.async_copy(
                data_v.at[pl.ds(j * 128, 128)],
                slab.at[ost2_v.at[j]],
                sem_add,
                add=True,
            ))
        for d in descs:
            d.wait()

    for s in range(NSLAB):
        zdescs = [
            pltpu.async_copy(
                zero_v, slab.at[pl.ds(sid * ZROWS + zi * SCHUNK, SCHUNK)], sem_g_a)
            for zi in range(zfull)
        ]
        zdescs.append(pltpu.async_copy(
            zero_v.at[pl.ds(0, zrem)],
            slab.at[pl.ds(sid * ZROWS + zfull * SCHUNK, zrem)],
            sem_g_a,
        ))
        for d in zdescs:
            d.wait()
        plsc.subcore_barrier()

        issue_lists(0, mst_a, ost_a, sem_l_a)

        def pair(i, carry):
            issue_lists(2 * i + 1, mst_b, ost_b, sem_l_b)
            drain_lists(mst_a, ost_a, sem_l_a)
            process(s, mst_a, ost_a, ost2_a, data_a, sem_g_a, sem_add_a)

            @pl.when(2 * i + 2 < n512)
            def _():
                issue_lists(2 * i + 2, mst_a, ost_a, sem_l_a)

            drain_lists(mst_b, ost_b, sem_l_b)
            process(s, mst_b, ost_b, ost2_b, data_b, sem_g_b, sem_add_b)
            return carry

        lax.fori_loop(0, n512 >> 1, pair, 0)
        plsc.subcore_barrier()
        pltpu.sync_copy(
            slab.at[pl.ds(sid * EXP_R, EXP_R)],
            out_hbm.at[pl.ds(rbase + sid * EXP_R, EXP_R), pl.ds(s * CS, CS)],
        )
        plsc.subcore_barrier()


@functools.partial(
    pl.kernel,
    out_type=jax.ShapeDtypeStruct((N_OUT_ROWS, C), jnp.float32),
    mesh=_MESH,
    scratch_types=[
        pltpu.VMEM_SHARED((SLAB_R, CS), jnp.float32),
        pltpu.VMEM((SCHUNK,), jnp.int32),
        pltpu.VMEM((SCHUNK,), jnp.int32),
        pltpu.VMEM((SCHUNK,), jnp.int32),
        pltpu.VMEM((SCHUNK,), jnp.int32),
        pltpu.VMEM((SCHUNK // 128, 128), jnp.int32),
        pltpu.VMEM((SCHUNK // 128, 128), jnp.int32),
        pltpu.VMEM((SCHUNK, CS), jnp.float32),
        pltpu.VMEM((SCHUNK, CS), jnp.float32),
        pltpu.VMEM((16,), jnp.int32),
        pltpu.VMEM((SCHUNK, CS), jnp.float32),
        pltpu.SemaphoreType.DMA,
        pltpu.SemaphoreType.DMA,
        pltpu.SemaphoreType.DMA,
        pltpu.SemaphoreType.DMA,
        pltpu.SemaphoreType.DMA,
        pltpu.SemaphoreType.DMA,
    ],
    compiler_params=_SC_PARAMS_NL,
)
def _scatter_call(m2_hbm, mlist_hbm, olist_hbm, cnt_hbm, out_hbm, slab,
                  mst_a, mst_b, ost_a, ost_b, ost2_a, ost2_b,
                  data_a, data_b, cvec, zero_v,
                  sem_l_a, sem_l_b, sem_g_a, sem_g_b, sem_add_a, sem_add_b):
    _scatter_body(m2_hbm, mlist_hbm, olist_hbm, cnt_hbm, out_hbm, slab,
                  mst_a, mst_b, ost_a, ost_b, ost2_a, ost2_b,
                  data_a, data_b, cvec, zero_v,
                  sem_l_a, sem_l_b, sem_g_a, sem_g_b, sem_add_a, sem_add_b)


def kernel(x, in_map, out_map, kernel):
    w = kernel
    pad = KP_PAD - KP
    im = in_map.reshape(-1).astype(jnp.int32)
    om = out_map.reshape(-1).astype(jnp.int32)
    pad_in = (jnp.arange(pad, dtype=jnp.int32) * 149) % N_IN_ROWS
    pad_out = N_OUT_ROWS + jnp.arange(pad, dtype=jnp.int32)
    im_p = jnp.concatenate([im, pad_in]).reshape(KP_PAD // 128, 128)
    om_p = jnp.concatenate([om, pad_out]).reshape(KP_PAD // 128, 128)

    mlist, olist, cnts = _partition_call(om_p)
    g = _gather_call(x, im_p)
    msgs = _matmul(g, w)
    m2 = msgs.reshape(KP_PAD * 8, CS)
    return _scatter_call(m2, mlist, olist, cnts)


# speedup vs baseline: 1.0056x; 1.0022x over previous
import functools

import jax
import jax.numpy as jnp
from jax import lax
from jax.experimental import pallas as pl
from jax.experimental.pallas import tpu as pltpu
from jax.experimental.pallas import tpu_sc as plsc

N_IN_ROWS = 100000
N_OUT_ROWS = 200000
NK = 27
NP = 20000
C = 128
KP = NK * NP

NCORE = 2
NSUB = 16
NW = NCORE * NSUB

ROWS_PER_W = 16896
KP_PAD = NW * ROWS_PER_W
GCHUNK = 384
NGCH = ROWS_PER_W // GCHUNK
GSTREAMS = GCHUNK // 128
IDX_ROWS_W = ROWS_PER_W // 128

MM_BLK = 2000

SCHUNK = 512
MS_PER_TILE = KP_PAD // NSUB
CS = 16
NSLAB = C // CS
HALF = N_OUT_ROWS // 2
TRASH = 1024
SLAB_R = HALF + TRASH
ZROWS = SLAB_R // NSUB
EXP_R = HALF // NSUB
LCAP = MS_PER_TILE + 1024
NST = MS_PER_TILE // 1024

_MESH = plsc.VectorSubcoreMesh(core_axis_name="c", subcore_axis_name="s")
_SC_PARAMS = pltpu.CompilerParams(use_tc_tiling_on_sc=False)
_SC_PARAMS_NL = pltpu.CompilerParams(use_tc_tiling_on_sc=False,
                                     needs_layout_passes=False)


def _gather_body(x_hbm, im_hbm, g_hbm, idx_v, rows_a, rows_b,
                 sem_ga, sem_gb, sem_wa, sem_wb):
    wid = lax.axis_index("s") * NCORE + lax.axis_index("c")
    pltpu.sync_copy(im_hbm.at[pl.ds(wid * IDX_ROWS_W, IDX_ROWS_W)], idx_v)
    base = wid * ROWS_PER_W

    def issue_g(ci, rows_v, sem):
        for j in range(GSTREAMS):
            pltpu.async_copy(
                x_hbm.at[idx_v.at[ci * GSTREAMS + j]],
                rows_v.at[pl.ds(j * 128, 128)],
                sem,
            )

    def drain_g(rows_v, sem):
        for j in range(GSTREAMS):
            pltpu.make_async_copy(
                x_hbm.at[pl.ds(0, 128)],
                rows_v.at[pl.ds(j * 128, 128)],
                sem,
            ).wait()

    def issue_w(ci, rows_v, sem):
        pltpu.async_copy(rows_v, g_hbm.at[pl.ds(base + ci * GCHUNK, GCHUNK)], sem)

    def drain_w(rows_v, sem):
        pltpu.make_async_copy(rows_v, g_hbm.at[pl.ds(0, GCHUNK)], sem).wait()

    issue_g(0, rows_a, sem_ga)

    def pair(i, carry):
        c = 2 * i
        drain_g(rows_a, sem_ga)

        @pl.when(i > 0)
        def _():
            drain_w(rows_b, sem_wb)

        issue_g(c + 1, rows_b, sem_gb)
        issue_w(c, rows_a, sem_wa)
        drain_g(rows_b, sem_gb)
        drain_w(rows_a, sem_wa)

        @pl.when(i < NGCH // 2 - 1)
        def _():
            issue_g(c + 2, rows_a, sem_ga)

        issue_w(c + 1, rows_b, sem_wb)
        return carry

    lax.fori_loop(0, NGCH // 2, pair, 0)
    drain_w(rows_b, sem_wb)


@functools.partial(
    pl.kernel,
    out_type=jax.ShapeDtypeStruct((KP_PAD, C), jnp.float32),
    mesh=_MESH,
    scratch_types=[
        pltpu.VMEM((IDX_ROWS_W, 128), jnp.int32),
        pltpu.VMEM((GCHUNK, C), jnp.float32),
        pltpu.VMEM((GCHUNK, C), jnp.float32),
        pltpu.SemaphoreType.DMA,
        pltpu.SemaphoreType.DMA,
        pltpu.SemaphoreType.DMA,
        pltpu.SemaphoreType.DMA,
    ],
    compiler_params=_SC_PARAMS,
)
def _gather_call(x_hbm, im_hbm, g_hbm, idx_v, rows_a, rows_b,
                 sem_ga, sem_gb, sem_wa, sem_wb):
    _gather_body(x_hbm, im_hbm, g_hbm, idx_v, rows_a, rows_b,
                 sem_ga, sem_gb, sem_wa, sem_wb)


def _mm_body(g_ref, w_ref, o_ref):
    o_ref[...] = jnp.dot(g_ref[...], w_ref[0], preferred_element_type=jnp.float32)


def _matmul(g, w):
    return pl.pallas_call(
        _mm_body,
        grid=(NK, NP // MM_BLK),
        in_specs=[
            pl.BlockSpec((MM_BLK, C), lambda k, b: (k * (NP // MM_BLK) + b, 0)),
            pl.BlockSpec((1, C, C), lambda k, b: (k, 0, 0)),
        ],
        out_specs=pl.BlockSpec((MM_BLK, C), lambda k, b: (k * (NP // MM_BLK) + b, 0)),
        out_shape=jax.ShapeDtypeStruct((KP_PAD, C), jnp.float32),
    )(g, w)


def _partition_body(om_hbm, mlist_hbm, olist_hbm, cnt_hbm,
                    stage_v, mbuf, obuf, cvec):
    cid = lax.axis_index("c")
    sid = lax.axis_index("s")
    rbase = cid * HALF
    lane = lax.iota(jnp.int32, 16)

    def staging(st, off):
        row0 = sid * (MS_PER_TILE // 128) + st * 8
        pltpu.sync_copy(om_hbm.at[pl.ds(row0, 8)], stage_v)
        gbase = sid * MS_PER_TILE + st * 1024
        for j in range(8):
            for i in range(8):
                v = stage_v[j, pl.ds(i * 16, 16)]
                r = v - rbase
                ok = (r >= 0) & (r < HALF)
                gidx = gbase + j * 128 + i * 16 + lane
                pos = plsc.cumsum(ok.astype(jnp.int32))
                dest = jnp.where(ok, off + pos - 1, LCAP + lane)
                plsc.store_scatter(mbuf, [dest], gidx)
                plsc.store_scatter(obuf, [dest], r)
                off = off + jnp.max(pos)
        return off

    off = lax.fori_loop(0, NST, staging, jnp.int32(0))
    for t in range(64):
        m_pad = lane + (t % 8) * 16
        o_pad = HALF + lane + (t % 8) * 16
        mbuf[pl.ds(off + t * 16, 16)] = m_pad
        obuf[pl.ds(off + t * 16, 16)] = o_pad
    padded = ((off + 1023) >> 10) << 10
    pvec = jnp.full((16,), 0, jnp.int32) + padded
    cvec[:] = jnp.maximum(pvec, 1024)
    pltpu.sync_copy(cvec, cnt_hbm.at[cid, sid])
    lbase = (cid * NSUB + sid) * LCAP
    pltpu.sync_copy(mbuf.at[pl.ds(0, LCAP)], mlist_hbm.at[pl.ds(lbase, LCAP)])
    pltpu.sync_copy(obuf.at[pl.ds(0, LCAP)], olist_hbm.at[pl.ds(lbase, LCAP)])


@functools.partial(
    pl.kernel,
    out_type=(
        jax.ShapeDtypeStruct((NCORE * NSUB * LCAP,), jnp.int32),
        jax.ShapeDtypeStruct((NCORE * NSUB * LCAP,), jnp.int32),
        jax.ShapeDtypeStruct((NCORE, NSUB, 16), jnp.int32),
    ),
    mesh=_MESH,
    scratch_types=[
        pltpu.VMEM((8, 128), jnp.int32),
        pltpu.VMEM((LCAP + 16,), jnp.int32),
        pltpu.VMEM((LCAP + 16,), jnp.int32),
        pltpu.VMEM((16,), jnp.int32),
    ],
    compiler_params=_SC_PARAMS_NL,
)
def _partition_call(om_hbm, mlist_hbm, olist_hbm, cnt_hbm,
                    stage_v, mbuf, obuf, cvec):
    _partition_body(om_hbm, mlist_hbm, olist_hbm, cnt_hbm,
                    stage_v, mbuf, obuf, cvec)


def _scatter_body(m2_hbm, mlist_hbm, olist_hbm, cnt_hbm, out_hbm, slab,
                  mst_a, mst_b, ost_a, ost_b, ost2_a, ost2_b,
                  data_a, data_b, cvec, zero_v,
                  sem_l_a, sem_l_b, sem_g_a, sem_g_b, sem_add_a, sem_add_b):
    cid = lax.axis_index("c")
    sid = lax.axis_index("s")
    rbase = cid * HALF
    lbase = (cid * NSUB + sid) * LCAP

    def zfill(i, carry):
        zero_v[i, :] = jnp.zeros((CS,), jnp.float32)
        return carry

    lax.fori_loop(0, SCHUNK, zfill, 0)
    zfull = ZROWS // SCHUNK
    zrem = ZROWS - zfull * SCHUNK

    pltpu.sync_copy(cnt_hbm.at[cid, sid], cvec)
    n512 = jnp.max(cvec[...]) >> 9

    def issue_lists(g, mst_v, ost_v, sem):
        pltpu.async_copy(mlist_hbm.at[pl.ds(lbase + g * 512, 512)], mst_v, sem)
        pltpu.async_copy(olist_hbm.at[pl.ds(lbase + g * 512, 512)], ost_v, sem)

    def drain_lists(mst_v, ost_v, sem):
        pltpu.make_async_copy(mlist_hbm.at[pl.ds(0, 512)], mst_v, sem).wait()
        pltpu.make_async_copy(olist_hbm.at[pl.ds(0, 512)], ost_v, sem).wait()

    def process(s, mst_v, ost_v, ost2_v, data_v, sem_g, sem_add):
        for k in range(32):
            m = mst_v[pl.ds(k * 16, 16)]
            mst_v[pl.ds(k * 16, 16)] = (m << 3) + s
            ost2_v[k // 8, pl.ds((k % 8) * 16, 16)] = ost_v[pl.ds(k * 16, 16)]
        descs = []
        for j in range(4):
            descs.append(pltpu.async_copy(
                m2_hbm.at[mst_v.at[pl.ds(j * 128, 128)]],
                data_v.at[pl.ds(j * 128, 128)],
                sem_g,
            ))
        for d in descs:
            d.wait()
        descs = []
        for j in range(4):
            descs.append(pltpu.async_copy(
                data_v.at[pl.ds(j * 128, 128)],
                slab.at[ost2_v.at[j]],
                sem_add,
                add=True,
            ))
        for d in descs:
            d.wait()

    for s in range(NSLAB):
        zdescs = [
            pltpu.async_copy(
                zero_v, slab.at[pl.ds(sid * ZROWS + zi * SCHUNK, SCHUNK)], sem_g_a)
            for zi in range(zfull)
        ]
        zdescs.append(pltpu.async_copy(
            zero_v.at[pl.ds(0, zrem)],
            slab.at[pl.ds(sid * ZROWS + zfull * SCHUNK, zrem)],
            sem_g_a,
        ))
        for d in zdescs:
            d.wait()
        plsc.subcore_barrier()

        issue_lists(0, mst_a, ost_a, sem_l_a)

        def pair(i, carry):
            issue_lists(2 * i + 1, mst_b, ost_b, sem_l_b)
            drain_lists(mst_a, ost_a, sem_l_a)
            process(s, mst_a, ost_a, ost2_a, data_a, sem_g_a, sem_add_a)

            @pl.when(2 * i + 2 < n512)
            def _():
                issue_lists(2 * i + 2, mst_a, ost_a, sem_l_a)

            drain_lists(mst_b, ost_b, sem_l_b)
            process(s, mst_b, ost_b, ost2_b, data_b, sem_g_b, sem_add_b)
            return carry

        lax.fori_loop(0, n512 >> 1, pair, 0)
        plsc.subcore_barrier()
        pltpu.sync_copy(
            slab.at[pl.ds(sid * EXP_R, EXP_R)],
            out_hbm.at[pl.ds(rbase + sid * EXP_R, EXP_R), pl.ds(s * CS, CS)],
        )
        plsc.subcore_barrier()


@functools.partial(
    pl.kernel,
    out_type=jax.ShapeDtypeStruct((N_OUT_ROWS, C), jnp.float32),
    mesh=_MESH,
    scratch_types=[
        pltpu.VMEM_SHARED((SLAB_R, CS), jnp.float32),
        pltpu.VMEM((SCHUNK,), jnp.int32),
        pltpu.VMEM((SCHUNK,), jnp.int32),
        pltpu.VMEM((SCHUNK,), jnp.int32),
        pltpu.VMEM((SCHUNK,), jnp.int32),
        pltpu.VMEM((SCHUNK // 128, 128), jnp.int32),
        pltpu.VMEM((SCHUNK // 128, 128), jnp.int32),
        pltpu.VMEM((SCHUNK, CS), jnp.float32),
        pltpu.VMEM((SCHUNK, CS), jnp.float32),
        pltpu.VMEM((16,), jnp.int32),
        pltpu.VMEM((SCHUNK, CS), jnp.float32),
        pltpu.SemaphoreType.DMA,
        pltpu.SemaphoreType.DMA,
        pltpu.SemaphoreType.DMA,
        pltpu.SemaphoreType.DMA,
        pltpu.SemaphoreType.DMA,
        pltpu.SemaphoreType.DMA,
    ],
    compiler_params=_SC_PARAMS_NL,
)
def _scatter_call(m2_hbm, mlist_hbm, olist_hbm, cnt_hbm, out_hbm, slab,
                  mst_a, mst_b, ost_a, ost_b, ost2_a, ost2_b,
                  data_a, data_b, cvec, zero_v,
                  sem_l_a, sem_l_b, sem_g_a, sem_g_b, sem_add_a, sem_add_b):
    _scatter_body(m2_hbm, mlist_hbm, olist_hbm, cnt_hbm, out_hbm, slab,
                  mst_a, mst_b, ost_a, ost_b, ost2_a, ost2_b,
                  data_a, data_b, cvec, zero_v,
                  sem_l_a, sem_l_b, sem_g_a, sem_g_b, sem_add_a, sem_add_b)


def kernel(x, in_map, out_map, kernel):
    w = kernel
    pad = KP_PAD - KP
    im = in_map.reshape(-1).astype(jnp.int32)
    om = out_map.reshape(-1).astype(jnp.int32)
    pad_in = (jnp.arange(pad, dtype=jnp.int32) * 149) % N_IN_ROWS
    pad_out = N_OUT_ROWS + jnp.arange(pad, dtype=jnp.int32)
    im_p = jnp.concatenate([im, pad_in]).reshape(KP_PAD // 128, 128)
    om_p = jnp.concatenate([om, pad_out]).reshape(KP_PAD // 128, 128)

    mlist, olist, cnts = _partition_call(om_p)
    g = _gather_call(x, im_p)
    msgs = _matmul(g, w)
    m2 = msgs.reshape(KP_PAD * 8, CS)
    return _scatter_call(m2, mlist, olist, cnts)
